# baseline (speedup 1.0000x reference)
import functools

import jax
import jax.numpy as jnp
from jax import lax
from jax.experimental import pallas as pl
from jax.experimental.pallas import tpu as pltpu
from jax.experimental.pallas import tpu_sc as plsc

NC = 2
NS = 16
NW = NC * NS
EK = 80
STG = 42
ZR = 32


def _sc_partial_sums(x2d, src4, dst4, n_pad):
    N, D = x2d.shape
    _, n_stages, stg, _ = src4.shape
    chunks = n_stages * stg
    rows_per_tile = n_pad // NS
    assert rows_per_tile % ZR == 0 and stg % 2 == 0

    mesh = plsc.VectorSubcoreMesh(
        core_axis_name="c", subcore_axis_name="s",
        num_cores=NC, num_subcores=NS)

    @functools.partial(
        pl.kernel,
        mesh=mesh,
        out_type=jax.ShapeDtypeStruct((NC, n_pad, D), jnp.float32),
        scratch_types=[
            pltpu.VMEM((stg, EK), jnp.int32),
            pltpu.VMEM((stg, EK), jnp.int32),
            pltpu.VMEM((EK, D), jnp.float32),
            pltpu.VMEM((EK, D), jnp.float32),
            pltpu.VMEM((ZR, D), jnp.float32),
            pltpu.VMEM_SHARED((n_pad, D), jnp.float32),
            pltpu.SemaphoreType.DMA,
            pltpu.SemaphoreType.DMA,
        ],
    )
    def k(x_hbm, src_hbm, dst_hbm, out_hbm,
          src_v, dst_v, rows_a, rows_b, zbuf, acc, sem_a, sem_b):
        c = lax.axis_index("c")
        s = lax.axis_index("s")
        wid = c * NS + s

        zero16 = jnp.zeros((16,), jnp.float32)

        @pl.loop(0, ZR)
        def _zero_row(i):
            for cc in range(D // 16):
                zbuf[i, pl.ds(cc * 16, 16)] = zero16

        row0 = s * rows_per_tile
        for z in range(rows_per_tile // ZR):
            pltpu.async_copy(zbuf, acc.at[pl.ds(row0 + z * ZR, ZR)], sem_a)
        for z in range(rows_per_tile // ZR):
            pltpu.make_async_copy(zbuf, acc.at[pl.ds(row0 + z * ZR, ZR)],
                                  sem_a).wait()
        plsc.subcore_barrier()

        @pl.loop(0, chunks // stg)
        def _stage(t):
            pltpu.sync_copy(src_hbm.at[wid, t], src_v)
            pltpu.sync_copy(dst_hbm.at[wid, t], dst_v)
            pltpu.async_copy(x_hbm.at[src_v.at[0]], rows_a, sem_a)

            @pl.loop(0, (stg - 2) // 2)
            def _pair(p):
                j = p * 2
                pltpu.async_copy(x_hbm.at[src_v.at[j + 1]], rows_b, sem_b)
                pltpu.make_async_copy(x_hbm.at[src_v.at[j]], rows_a,
                                      sem_a).wait()
                pltpu.sync_copy(rows_a, acc.at[dst_v.at[j]], add=True)
                pltpu.async_copy(x_hbm.at[src_v.at[j + 2]], rows_a, sem_a)
                pltpu.make_async_copy(x_hbm.at[src_v.at[j + 1]], rows_b,
                                      sem_b).wait()
                pltpu.sync_copy(rows_b, acc.at[dst_v.at[j + 1]], add=True)

            pltpu.async_copy(x_hbm.at[src_v.at[stg - 1]], rows_b, sem_b)
            pltpu.make_async_copy(x_hbm.at[src_v.at[stg - 2]], rows_a,
                                  sem_a).wait()
            pltpu.sync_copy(rows_a, acc.at[dst_v.at[stg - 2]], add=True)
            pltpu.make_async_copy(x_hbm.at[src_v.at[stg - 1]], rows_b,
                                  sem_b).wait()
            pltpu.sync_copy(rows_b, acc.at[dst_v.at[stg - 1]], add=True)

        plsc.subcore_barrier()
        pltpu.sync_copy(acc.at[pl.ds(row0, rows_per_tile)],
                        out_hbm.at[c, pl.ds(row0, rows_per_tile)])

    return k(x2d, src4, dst4)


def _ln(h, g, b):
    m = jnp.mean(h, axis=-1, keepdims=True)
    d = h - m
    v = jnp.mean(d * d, axis=-1, keepdims=True)
    return d * lax.rsqrt(v + 1e-5) * g + b


def _dense1_body(x_ref, g1_ref, b1g_ref, w1a_ref, b1_ref, t1_ref):
    h1 = _ln(x_ref[...], g1_ref[...], b1g_ref[...])
    t1_ref[...] = (jnp.dot(h1, w1a_ref[...],
                           preferred_element_type=jnp.float32) + b1_ref[...])


def _dense2_body(x_ref, t1_ref, part_ref, deg_ref, g2_ref, b2g_ref,
                 w1b_ref, w2_ref, b2_ref, out_ref):
    neigh = (part_ref[0] + part_ref[1]) * (
        1.0 / jnp.clip(deg_ref[...], 1.0, None))
    h2 = _ln(neigh, g2_ref[...], b2g_ref[...])
    h = t1_ref[...] + jnp.dot(h2, w1b_ref[...],
                              preferred_element_type=jnp.float32)
    h = 0.5 * h * (1.0 + lax.erf(h * 0.7071067811865476))
    out_ref[...] = (x_ref[...] + jnp.dot(h, w2_ref[...],
                                         preferred_element_type=jnp.float32)
                    + b2_ref[...])


def _full(a):
    return pl.BlockSpec(a.shape, lambda i: tuple(0 for _ in a.shape))


def _dense1(x2d, g1, b1g, w1a, b1):
    N, D = x2d.shape
    BN = 1000
    row_blk = pl.BlockSpec((BN, D), lambda i: (i, 0))
    t1_blk = pl.BlockSpec((BN, 2 * D), lambda i: (i, 0))
    return pl.pallas_call(
        _dense1_body,
        grid=(N // BN,),
        in_specs=[row_blk, _full(g1), _full(b1g), _full(w1a), _full(b1)],
        out_specs=t1_blk,
        out_shape=jax.ShapeDtypeStruct((N, 2 * D), jnp.float32),
    )(x2d, g1, b1g, w1a, b1)


def _dense2(x2d, t1, part, deg_col, g2, b2g, w1b, w2t, b2):
    N, D = x2d.shape
    BN = 1000
    row_blk = pl.BlockSpec((BN, D), lambda i: (i, 0))
    t1_blk = pl.BlockSpec((BN, 2 * D), lambda i: (i, 0))
    part_blk = pl.BlockSpec((2, BN, D), lambda i: (0, i, 0))
    col_blk = pl.BlockSpec((BN, 1), lambda i: (i, 0))
    return pl.pallas_call(
        _dense2_body,
        grid=(N // BN,),
        in_specs=[row_blk, t1_blk, part_blk, col_blk,
                  _full(g2), _full(b2g), _full(w1b), _full(w2t), _full(b2)],
        out_specs=row_blk,
        out_shape=jax.ShapeDtypeStruct((N, D), jnp.float32),
    )(x2d, t1, part, deg_col, g2, b2g, w1b, w2t, b2)


def kernel(x, edge_src, edge_dst, degree,
           ln1_g, ln1_b, ln2_g, ln2_b, W1, b1, W2, b2):
    B, N, D = x.shape
    E = edge_src.shape[0]
    x2d = x.reshape(N, D)

    n_pad = ((N + NS * 128 - 1) // (NS * 128)) * (NS * 128)

    e_tile = E // NW
    chunks = -(-e_tile // (STG * EK)) * STG
    pad = chunks * EK - e_tile
    src2 = edge_src.astype(jnp.int32).reshape(NW, e_tile)
    dst2 = edge_dst.astype(jnp.int32).reshape(NW, e_tile)
    if pad > 0:
        spare = n_pad - N
        ar = jnp.arange(NW * pad, dtype=jnp.int32)
        src2 = jnp.concatenate(
            [src2, (ar % N).reshape(NW, pad)], axis=1)
        dst2 = jnp.concatenate(
            [dst2, (N + ar % spare).reshape(NW, pad)], axis=1)
    src4 = src2.reshape(NW, chunks // STG, STG, EK)
    dst4 = dst2.reshape(NW, chunks // STG, STG, EK)

    deg_col = degree.reshape(N, 1)
    w1a = W1[:, :D].T
    w1b = W1[:, D:].T
    w2t = W2.T

    t1 = _dense1(x2d, ln1_g.reshape(1, D), ln1_b.reshape(1, D),
                 w1a, b1.reshape(1, 2 * D))
    part = _sc_partial_sums(x2d, src4, dst4, n_pad)
    out2d = _dense2(x2d, t1, part, deg_col,
                    ln2_g.reshape(1, D), ln2_b.reshape(1, D),
                    w1b, w2t, b2.reshape(1, D))
    return out2d.reshape(B, N, D)


# speedup vs baseline: 1.0190x; 1.0190x over previous
import functools

import jax
import jax.numpy as jnp
from jax import lax
from jax.experimental import pallas as pl
from jax.experimental.pallas import tpu as pltpu
from jax.experimental.pallas import tpu_sc as plsc

NC = 2
NS = 16
NW = NC * NS
EK = 80
STG = 42
ZR = 32


def _sc_partial_sums(x2d, src4, dst4, n_pad):
    N, D = x2d.shape
    _, n_stages, stg, _ = src4.shape
    chunks = n_stages * stg
    rows_per_tile = n_pad // NS
    assert rows_per_tile % ZR == 0 and stg % 2 == 0

    mesh = plsc.VectorSubcoreMesh(
        core_axis_name="c", subcore_axis_name="s",
        num_cores=NC, num_subcores=NS)

    @functools.partial(
        pl.kernel,
        mesh=mesh,
        out_type=jax.ShapeDtypeStruct((NC, n_pad, D), jnp.float32),
        scratch_types=[
            pltpu.VMEM((stg, EK), jnp.int32),
            pltpu.VMEM((stg, EK), jnp.int32),
            pltpu.VMEM((EK, D), jnp.float32),
            pltpu.VMEM((EK, D), jnp.float32),
            pltpu.VMEM((ZR, D), jnp.float32),
            pltpu.VMEM_SHARED((n_pad, D), jnp.float32),
            pltpu.SemaphoreType.DMA,
            pltpu.SemaphoreType.DMA,
        ],
    )
    def k(x_hbm, src_hbm, dst_hbm, out_hbm,
          src_v, dst_v, rows_a, rows_b, zbuf, acc, sem_a, sem_b):
        c = lax.axis_index("c")
        s = lax.axis_index("s")
        wid = c * NS + s

        zero16 = jnp.zeros((16,), jnp.float32)

        @pl.loop(0, ZR)
        def _zero_row(i):
            for cc in range(D // 16):
                zbuf[i, pl.ds(cc * 16, 16)] = zero16

        row0 = s * rows_per_tile
        for z in range(rows_per_tile // ZR):
            pltpu.async_copy(zbuf, acc.at[pl.ds(row0 + z * ZR, ZR)], sem_a)
        for z in range(rows_per_tile // ZR):
            pltpu.make_async_copy(zbuf, acc.at[pl.ds(row0 + z * ZR, ZR)],
                                  sem_a).wait()
        plsc.subcore_barrier()

        @pl.loop(0, chunks // stg)
        def _stage(t):
            pltpu.sync_copy(src_hbm.at[wid, t], src_v)
            pltpu.sync_copy(dst_hbm.at[wid, t], dst_v)
            pltpu.async_copy(x_hbm.at[src_v.at[0]], rows_a, sem_a)

            @pl.loop(0, (stg - 2) // 2)
            def _pair(p):
                j = p * 2
                pltpu.async_copy(x_hbm.at[src_v.at[j + 1]], rows_b, sem_b)
                pltpu.make_async_copy(x_hbm.at[src_v.at[j]], rows_a,
                                      sem_a).wait()
                pltpu.sync_copy(rows_a, acc.at[dst_v.at[j]], add=True)
                pltpu.async_copy(x_hbm.at[src_v.at[j + 2]], rows_a, sem_a)
                pltpu.make_async_copy(x_hbm.at[src_v.at[j + 1]], rows_b,
                                      sem_b).wait()
                pltpu.sync_copy(rows_b, acc.at[dst_v.at[j + 1]], add=True)

            pltpu.async_copy(x_hbm.at[src_v.at[stg - 1]], rows_b, sem_b)
            pltpu.make_async_copy(x_hbm.at[src_v.at[stg - 2]], rows_a,
                                  sem_a).wait()
            pltpu.sync_copy(rows_a, acc.at[dst_v.at[stg - 2]], add=True)
            pltpu.make_async_copy(x_hbm.at[src_v.at[stg - 1]], rows_b,
                                  sem_b).wait()
            pltpu.sync_copy(rows_b, acc.at[dst_v.at[stg - 1]], add=True)

        plsc.subcore_barrier()
        pltpu.sync_copy(acc.at[pl.ds(row0, rows_per_tile)],
                        out_hbm.at[c, pl.ds(row0, rows_per_tile)])

    return k(x2d, src4, dst4)


def _ln(h, g, b):
    m = jnp.mean(h, axis=-1, keepdims=True)
    d = h - m
    v = jnp.mean(d * d, axis=-1, keepdims=True)
    return d * lax.rsqrt(v + 1e-5) * g + b


def _dense_body(x_ref, part_ref, deg_ref,
                g1_ref, b1g_ref, g2_ref, b2g_ref,
                w1a_ref, w1b_ref, b1_ref, w2_ref, b2_ref, out_ref):
    xb = x_ref[...]
    neigh = (part_ref[0] + part_ref[1]) * (
        1.0 / jnp.clip(deg_ref[...], 1.0, None))
    h1 = _ln(xb, g1_ref[...], b1g_ref[...])
    h2 = _ln(neigh, g2_ref[...], b2g_ref[...])
    h = (jnp.dot(h1, w1a_ref[...], preferred_element_type=jnp.float32)
         + jnp.dot(h2, w1b_ref[...], preferred_element_type=jnp.float32)
         + b1_ref[...])
    h = 0.5 * h * (1.0 + lax.erf(h * 0.7071067811865476))
    out_ref[...] = (xb + jnp.dot(h, w2_ref[...],
                                 preferred_element_type=jnp.float32)
                    + b2_ref[...])


def _full(a):
    return pl.BlockSpec(a.shape, lambda i: tuple(0 for _ in a.shape))


def _dense(x2d, part, deg_col, g1, b1g, g2, b2g, w1a, w1b, b1, w2t, b2):
    N, D = x2d.shape
    BN = 1000
    row_blk = pl.BlockSpec((BN, D), lambda i: (i, 0))
    part_blk = pl.BlockSpec((2, BN, D), lambda i: (0, i, 0))
    col_blk = pl.BlockSpec((BN, 1), lambda i: (i, 0))
    return pl.pallas_call(
        _dense_body,
        grid=(N // BN,),
        in_specs=[row_blk, part_blk, col_blk,
                  _full(g1), _full(b1g), _full(g2), _full(b2g),
                  _full(w1a), _full(w1b), _full(b1), _full(w2t), _full(b2)],
        out_specs=row_blk,
        out_shape=jax.ShapeDtypeStruct((N, D), jnp.float32),
    )(x2d, part, deg_col, g1, b1g, g2, b2g, w1a, w1b, b1, w2t, b2)


def kernel(x, edge_src, edge_dst, degree,
           ln1_g, ln1_b, ln2_g, ln2_b, W1, b1, W2, b2):
    B, N, D = x.shape
    E = edge_src.shape[0]
    x2d = x.reshape(N, D)

    n_pad = ((N + NS * 128 - 1) // (NS * 128)) * (NS * 128)

    e_tile = E // NW
    chunks = -(-e_tile // (STG * EK)) * STG
    pad = chunks * EK - e_tile
    src2 = edge_src.astype(jnp.int32).reshape(NW, e_tile)
    dst2 = edge_dst.astype(jnp.int32).reshape(NW, e_tile)
    if pad > 0:
        spare = n_pad - N
        ar = jnp.arange(NW * pad, dtype=jnp.int32)
        src2 = jnp.concatenate(
            [src2, (ar % N).reshape(NW, pad)], axis=1)
        dst2 = jnp.concatenate(
            [dst2, (N + ar % spare).reshape(NW, pad)], axis=1)
    src4 = src2.reshape(NW, chunks // STG, STG, EK)
    dst4 = dst2.reshape(NW, chunks // STG, STG, EK)

    part = _sc_partial_sums(x2d, src4, dst4, n_pad)

    deg_col = degree.reshape(N, 1)
    w1a = W1[:, :D].T
    w1b = W1[:, D:].T
    w2t = W2.T
    out2d = _dense(x2d, part, deg_col,
                   ln1_g.reshape(1, D), ln1_b.reshape(1, D),
                   ln2_g.reshape(1, D), ln2_b.reshape(1, D),
                   w1a, w1b, b1.reshape(1, 2 * D), w2t, b2.reshape(1, D))
    return out2d.reshape(B, N, D)


# speedup vs baseline: 1.0548x; 1.0351x over previous
import functools

import jax
import jax.numpy as jnp
from jax import lax
from jax.experimental import pallas as pl
from jax.experimental.pallas import tpu as pltpu
from jax.experimental.pallas import tpu_sc as plsc

NC = 2
NS = 16
NW = NC * NS
EK = 80
STG = 42
ZR = 32


def _sc_partial_sums(x2d, src4, dst4, n_pad):
    N, D = x2d.shape
    _, n_stages, stg, _ = src4.shape
    chunks = n_stages * stg
    rows_per_tile = n_pad // NS
    assert rows_per_tile % ZR == 0 and stg % 2 == 0

    mesh = plsc.VectorSubcoreMesh(
        core_axis_name="c", subcore_axis_name="s",
        num_cores=NC, num_subcores=NS)

    @functools.partial(
        pl.kernel,
        mesh=mesh,
        out_type=jax.ShapeDtypeStruct((NC, n_pad, D), jnp.float32),
        scratch_types=[
            pltpu.VMEM((stg, EK), jnp.int32),
            pltpu.VMEM((stg, EK), jnp.int32),
            pltpu.VMEM((EK, D), jnp.float32),
            pltpu.VMEM((EK, D), jnp.float32),
            pltpu.VMEM((ZR, D), jnp.float32),
            pltpu.VMEM_SHARED((n_pad, D), jnp.float32),
            pltpu.SemaphoreType.DMA,
            pltpu.SemaphoreType.DMA,
        ],
    )
    def k(x_hbm, src_hbm, dst_hbm, out_hbm,
          src_v, dst_v, rows_a, rows_b, zbuf, acc, sem_a, sem_b):
        c = lax.axis_index("c")
        s = lax.axis_index("s")
        wid = c * NS + s

        zero16 = jnp.zeros((16,), jnp.float32)

        @pl.loop(0, ZR)
        def _zero_row(i):
            for cc in range(D // 16):
                zbuf[i, pl.ds(cc * 16, 16)] = zero16

        row0 = s * rows_per_tile
        for z in range(rows_per_tile // ZR):
            pltpu.async_copy(zbuf, acc.at[pl.ds(row0 + z * ZR, ZR)], sem_a)
        for z in range(rows_per_tile // ZR):
            pltpu.make_async_copy(zbuf, acc.at[pl.ds(row0 + z * ZR, ZR)],
                                  sem_a).wait()
        plsc.subcore_barrier()

        @pl.loop(0, chunks // stg)
        def _stage(t):
            pltpu.sync_copy(src_hbm.at[wid, t], src_v)
            pltpu.sync_copy(dst_hbm.at[wid, t], dst_v)
            pltpu.async_copy(x_hbm.at[src_v.at[0]], rows_a, sem_a)

            @pl.loop(0, (stg - 2) // 2)
            def _pair(p):
                j = p * 2
                pltpu.async_copy(x_hbm.at[src_v.at[j + 1]], rows_b, sem_b)
                pltpu.make_async_copy(x_hbm.at[src_v.at[j]], rows_a,
                                      sem_a).wait()
                pltpu.sync_copy(rows_a, acc.at[dst_v.at[j]], add=True)
                pltpu.async_copy(x_hbm.at[src_v.at[j + 2]], rows_a, sem_a)
                pltpu.make_async_copy(x_hbm.at[src_v.at[j + 1]], rows_b,
                                      sem_b).wait()
                pltpu.sync_copy(rows_b, acc.at[dst_v.at[j + 1]], add=True)

            pltpu.async_copy(x_hbm.at[src_v.at[stg - 1]], rows_b, sem_b)
            pltpu.make_async_copy(x_hbm.at[src_v.at[stg - 2]], rows_a,
                                  sem_a).wait()
            pltpu.sync_copy(rows_a, acc.at[dst_v.at[stg - 2]], add=True)
            pltpu.make_async_copy(x_hbm.at[src_v.at[stg - 1]], rows_b,
                                  sem_b).wait()
            pltpu.sync_copy(rows_b, acc.at[dst_v.at[stg - 1]], add=True)

        plsc.subcore_barrier()
        pltpu.sync_copy(acc.at[pl.ds(row0, rows_per_tile)],
                        out_hbm.at[c, pl.ds(row0, rows_per_tile)])

    return k(x2d, src4, dst4)


def _ln(h, g, b):
    m = jnp.mean(h, axis=-1, keepdims=True)
    d = h - m
    v = jnp.mean(d * d, axis=-1, keepdims=True)
    return d * lax.rsqrt(v + 1e-5) * g + b


def _dense_body(x_ref, part_ref, deg_ref,
                g1_ref, b1g_ref, g2_ref, b2g_ref,
                w1a_ref, w1b_ref, b1_ref, w2_ref, b2_ref, out_ref):
    xb = x_ref[...]
    neigh = (part_ref[0] + part_ref[1]) * (
        1.0 / jnp.clip(deg_ref[...], 1.0, None))
    h1 = _ln(xb, g1_ref[...], b1g_ref[...])
    h2 = _ln(neigh, g2_ref[...], b2g_ref[...])
    h = (jnp.dot(h1, w1a_ref[...], preferred_element_type=jnp.float32)
         + jnp.dot(h2, w1b_ref[...], preferred_element_type=jnp.float32)
         + b1_ref[...])
    h = 0.5 * h * (1.0 + lax.erf(h * 0.7071067811865476))
    out_ref[...] = (xb + jnp.dot(h, w2_ref[...],
                                 preferred_element_type=jnp.float32)
                    + b2_ref[...])


def _full(a):
    return pl.BlockSpec(a.shape, lambda i: tuple(0 for _ in a.shape))


def _dense(x2d, part, deg_col, g1, b1g, g2, b2g, w1a, w1b, b1, w2t, b2):
    N, D = x2d.shape
    BN = 2000
    row_blk = pl.BlockSpec((BN, D), lambda i: (i, 0))
    part_blk = pl.BlockSpec((2, BN, D), lambda i: (0, i, 0))
    col_blk = pl.BlockSpec((BN, 1), lambda i: (i, 0))
    return pl.pallas_call(
        _dense_body,
        grid=(N // BN,),
        in_specs=[row_blk, part_blk, col_blk,
                  _full(g1), _full(b1g), _full(g2), _full(b2g),
                  _full(w1a), _full(w1b), _full(b1), _full(w2t), _full(b2)],
        out_specs=row_blk,
        out_shape=jax.ShapeDtypeStruct((N, D), jnp.float32),
    )(x2d, part, deg_col, g1, b1g, g2, b2g, w1a, w1b, b1, w2t, b2)


def kernel(x, edge_src, edge_dst, degree,
           ln1_g, ln1_b, ln2_g, ln2_b, W1, b1, W2, b2):
    B, N, D = x.shape
    E = edge_src.shape[0]
    x2d = x.reshape(N, D)

    n_pad = ((N + NS * 128 - 1) // (NS * 128)) * (NS * 128)

    e_tile = E // NW
    chunks = -(-e_tile // (STG * EK)) * STG
    pad = chunks * EK - e_tile
    src2 = edge_src.astype(jnp.int32).reshape(NW, e_tile)
    dst2 = edge_dst.astype(jnp.int32).reshape(NW, e_tile)
    if pad > 0:
        spare = n_pad - N
        ar = jnp.arange(NW * pad, dtype=jnp.int32)
        src2 = jnp.concatenate(
            [src2, (ar % N).reshape(NW, pad)], axis=1)
        dst2 = jnp.concatenate(
            [dst2, (N + ar % spare).reshape(NW, pad)], axis=1)
    src4 = src2.reshape(NW, chunks // STG, STG, EK)
    dst4 = dst2.reshape(NW, chunks // STG, STG, EK)

    part = _sc_partial_sums(x2d, src4, dst4, n_pad)

    deg_col = degree.reshape(N, 1)
    w1a = W1[:, :D].T
    w1b = W1[:, D:].T
    w2t = W2.T
    out2d = _dense(x2d, part, deg_col,
                   ln1_g.reshape(1, D), ln1_b.reshape(1, D),
                   ln2_g.reshape(1, D), ln2_b.reshape(1, D),
                   w1a, w1b, b1.reshape(1, 2 * D), w2t, b2.reshape(1, D))
    return out2d.reshape(B, N, D)


# speedup vs baseline: 1.0566x; 1.0017x over previous
import functools

import jax
import jax.numpy as jnp
from jax import lax
from jax.experimental import pallas as pl
from jax.experimental.pallas import tpu as pltpu
from jax.experimental.pallas import tpu_sc as plsc

NC = 2
NS = 16
NW = NC * NS
EK = 80
STG = 42
ZR = 32


def _sc_partial_sums(x2d, src4, dst4, n_pad):
    N, D = x2d.shape
    _, n_stages, stg, _ = src4.shape
    chunks = n_stages * stg
    rows_per_tile = n_pad // NS
    assert rows_per_tile % ZR == 0 and stg % 2 == 0

    mesh = plsc.VectorSubcoreMesh(
        core_axis_name="c", subcore_axis_name="s",
        num_cores=NC, num_subcores=NS)

    @functools.partial(
        pl.kernel,
        mesh=mesh,
        out_type=jax.ShapeDtypeStruct((NC, n_pad, D), jnp.float32),
        scratch_types=[
            pltpu.VMEM((stg, EK), jnp.int32),
            pltpu.VMEM((stg, EK), jnp.int32),
            pltpu.VMEM((EK, D), jnp.float32),
            pltpu.VMEM((EK, D), jnp.float32),
            pltpu.VMEM((ZR, D), jnp.float32),
            pltpu.VMEM_SHARED((n_pad, D), jnp.float32),
            pltpu.SemaphoreType.DMA,
            pltpu.SemaphoreType.DMA,
        ],
    )
    def k(x_hbm, src_hbm, dst_hbm, out_hbm,
          src_v, dst_v, rows_a, rows_b, zbuf, acc, sem_a, sem_b):
        c = lax.axis_index("c")
        s = lax.axis_index("s")
        wid = c * NS + s

        zero16 = jnp.zeros((16,), jnp.float32)

        @pl.loop(0, ZR)
        def _zero_row(i):
            for cc in range(D // 16):
                zbuf[i, pl.ds(cc * 16, 16)] = zero16

        row0 = s * rows_per_tile
        for z in range(rows_per_tile // ZR):
            pltpu.async_copy(zbuf, acc.at[pl.ds(row0 + z * ZR, ZR)], sem_a)
        for z in range(rows_per_tile // ZR):
            pltpu.make_async_copy(zbuf, acc.at[pl.ds(row0 + z * ZR, ZR)],
                                  sem_a).wait()
        plsc.subcore_barrier()

        @pl.loop(0, chunks // stg)
        def _stage(t):
            pltpu.sync_copy(src_hbm.at[wid, t], src_v)
            pltpu.sync_copy(dst_hbm.at[wid, t], dst_v)
            pltpu.async_copy(x_hbm.at[src_v.at[0]], rows_a, sem_a)

            @pl.loop(0, (stg - 2) // 2)
            def _pair(p):
                j = p * 2
                pltpu.async_copy(x_hbm.at[src_v.at[j + 1]], rows_b, sem_b)
                pltpu.make_async_copy(x_hbm.at[src_v.at[j]], rows_a,
                                      sem_a).wait()
                pltpu.sync_copy(rows_a, acc.at[dst_v.at[j]], add=True)
                pltpu.async_copy(x_hbm.at[src_v.at[j + 2]], rows_a, sem_a)
                pltpu.make_async_copy(x_hbm.at[src_v.at[j + 1]], rows_b,
                                      sem_b).wait()
                pltpu.sync_copy(rows_b, acc.at[dst_v.at[j + 1]], add=True)

            pltpu.async_copy(x_hbm.at[src_v.at[stg - 1]], rows_b, sem_b)
            pltpu.make_async_copy(x_hbm.at[src_v.at[stg - 2]], rows_a,
                                  sem_a).wait()
            pltpu.sync_copy(rows_a, acc.at[dst_v.at[stg - 2]], add=True)
            pltpu.make_async_copy(x_hbm.at[src_v.at[stg - 1]], rows_b,
                                  sem_b).wait()
            pltpu.sync_copy(rows_b, acc.at[dst_v.at[stg - 1]], add=True)

        plsc.subcore_barrier()
        pltpu.sync_copy(acc.at[pl.ds(row0, rows_per_tile)],
                        out_hbm.at[c, pl.ds(row0, rows_per_tile)])

    return k(x2d, src4, dst4)


def _ln(h, g, b):
    m = jnp.mean(h, axis=-1, keepdims=True)
    d = h - m
    v = jnp.mean(d * d, axis=-1, keepdims=True)
    return d * lax.rsqrt(v + 1e-5) * g + b


def _dense_body(x_ref, part_ref, deg_ref,
                g1_ref, b1g_ref, g2_ref, b2g_ref,
                w1a_ref, w1b_ref, b1_ref, w2_ref, b2_ref, out_ref):
    xb = x_ref[...]
    neigh = (part_ref[0] + part_ref[1]) * (
        1.0 / jnp.clip(deg_ref[...], 1.0, None))
    h1 = _ln(xb, g1_ref[...], b1g_ref[...])
    h2 = _ln(neigh, g2_ref[...], b2g_ref[...])
    h = (jnp.dot(h1, w1a_ref[...], preferred_element_type=jnp.float32)
         + jnp.dot(h2, w1b_ref[...], preferred_element_type=jnp.float32)
         + b1_ref[...])
    h = 0.5 * h * (1.0 + lax.erf(h * 0.7071067811865476))
    out_ref[...] = (xb + jnp.dot(h, w2_ref[...],
                                 preferred_element_type=jnp.float32)
                    + b2_ref[...])


def _full(a):
    return pl.BlockSpec(a.shape, lambda i: tuple(0 for _ in a.shape))


def _dense(x2d, part, deg_col, g1, b1g, g2, b2g, w1a, w1b, b1, w2t, b2):
    N, D = x2d.shape
    BN = 5000
    row_blk = pl.BlockSpec((BN, D), lambda i: (i, 0))
    part_blk = pl.BlockSpec((2, BN, D), lambda i: (0, i, 0))
    col_blk = pl.BlockSpec((BN, 1), lambda i: (i, 0))
    return pl.pallas_call(
        _dense_body,
        grid=(N // BN,),
        in_specs=[row_blk, part_blk, col_blk,
                  _full(g1), _full(b1g), _full(g2), _full(b2g),
                  _full(w1a), _full(w1b), _full(b1), _full(w2t), _full(b2)],
        out_specs=row_blk,
        out_shape=jax.ShapeDtypeStruct((N, D), jnp.float32),
    )(x2d, part, deg_col, g1, b1g, g2, b2g, w1a, w1b, b1, w2t, b2)


def kernel(x, edge_src, edge_dst, degree,
           ln1_g, ln1_b, ln2_g, ln2_b, W1, b1, W2, b2):
    B, N, D = x.shape
    E = edge_src.shape[0]
    x2d = x.reshape(N, D)

    n_pad = ((N + NS * 128 - 1) // (NS * 128)) * (NS * 128)

    e_tile = E // NW
    chunks = -(-e_tile // (STG * EK)) * STG
    pad = chunks * EK - e_tile
    src2 = edge_src.astype(jnp.int32).reshape(NW, e_tile)
    dst2 = edge_dst.astype(jnp.int32).reshape(NW, e_tile)
    if pad > 0:
        spare = n_pad - N
        ar = jnp.arange(NW * pad, dtype=jnp.int32)
        src2 = jnp.concatenate(
            [src2, (ar % N).reshape(NW, pad)], axis=1)
        dst2 = jnp.concatenate(
            [dst2, (N + ar % spare).reshape(NW, pad)], axis=1)
    src4 = src2.reshape(NW, chunks // STG, STG, EK)
    dst4 = dst2.reshape(NW, chunks // STG, STG, EK)

    part = _sc_partial_sums(x2d, src4, dst4, n_pad)

    deg_col = degree.reshape(N, 1)
    w1a = W1[:, :D].T
    w1b = W1[:, D:].T
    w2t = W2.T
    out2d = _dense(x2d, part, deg_col,
                   ln1_g.reshape(1, D), ln1_b.reshape(1, D),
                   ln2_g.reshape(1, D), ln2_b.reshape(1, D),
                   w1a, w1b, b1.reshape(1, 2 * D), w2t, b2.reshape(1, D))
    return out2d.reshape(B, N, D)
